# baseline (speedup 1.0000x reference)
import functools
import math

import jax
import jax.numpy as jnp
import ml_dtypes
import numpy as np
from jax import lax
from jax.experimental import pallas as pl
from jax.experimental.pallas import tpu as pltpu
from jax.experimental.pallas import tpu_sc as plsc

VOCAB = 100000
D_MODEL = 768
BATCH = 4
SEQ_LEN = 4096
SCALE = math.sqrt(D_MODEL)

NUM_CORES = 2
NUM_SUBCORES = 16
NW = NUM_CORES * NUM_SUBCORES
POS_PER_W = SEQ_LEN // NW
CHUNK = 32
N_PCHUNK = POS_PER_W // CHUNK
NCHUNK = N_PCHUNK * BATCH
LANES = 16
VPR = D_MODEL // LANES
NBUF = 3


def _pos_encoding() -> np.ndarray:
    pos = np.arange(SEQ_LEN)[:, None].astype(np.float32)
    div = np.exp(
        np.arange(0, D_MODEL, 2).astype(np.float32)
        * (-math.log(10000.0) / D_MODEL)
    )
    pe = np.zeros((SEQ_LEN, D_MODEL), dtype=np.float32)
    pe[:, 0::2] = np.sin(pos * div)
    pe[:, 1::2] = np.cos(pos * div)
    pe_bits = pe.astype(ml_dtypes.bfloat16).view(np.uint16).astype(np.uint32)
    pe4 = pe_bits.reshape(SEQ_LEN, D_MODEL // 32, 2, 16)
    packed = pe4[:, :, 0, :] | (pe4[:, :, 1, :] << 16)
    return packed.reshape(SEQ_LEN * (D_MODEL // 2)).view(np.int32)


_PE = _pos_encoding()


@functools.partial(
    pl.kernel,
    out_type=jax.ShapeDtypeStruct((BATCH * SEQ_LEN, D_MODEL), jnp.float32),
    mesh=plsc.VectorSubcoreMesh(core_axis_name="c", subcore_axis_name="s"),
    compiler_params=pltpu.CompilerParams(needs_layout_passes=False),
    scratch_types=[
        pltpu.VMEM((BATCH, POS_PER_W), jnp.int32),
        pltpu.VMEM((CHUNK * D_MODEL // 2,), jnp.int32),
        pltpu.VMEM((CHUNK * D_MODEL // 2,), jnp.int32),
        pltpu.VMEM((CHUNK, D_MODEL), jnp.float32),
        pltpu.VMEM((CHUNK, D_MODEL), jnp.float32),
        pltpu.VMEM((CHUNK, D_MODEL), jnp.float32),
        pltpu.SemaphoreType.DMA,
        pltpu.SemaphoreType.DMA,
        pltpu.SemaphoreType.DMA,
        pltpu.SemaphoreType.DMA,
        pltpu.SemaphoreType.DMA,
        pltpu.SemaphoreType.DMA,
        pltpu.SemaphoreType.DMA,
        pltpu.SemaphoreType.DMA,
    ],
)
def _embed(ids_hbm, table_hbm, pe_hbm, out_hbm,
           idx_all, pe0, pe1, r0, r1, r2,
           g0, g1, g2, w0, w1, w2, p0, p1):
    wid = lax.axis_index("s") * NUM_CORES + lax.axis_index("c")
    pos_base = wid * POS_PER_W

    rows = (r0, r1, r2)
    gsem = (g0, g1, g2)
    wsem = (w0, w1, w2)
    pebuf = (pe0, pe1)
    psem = (p0, p1)

    pltpu.sync_copy(ids_hbm.at[:, pl.ds(pos_base, POS_PER_W)], idx_all)
    pltpu.sync_copy(
        pe_hbm.at[pl.ds(pos_base * (D_MODEL // 2), CHUNK * D_MODEL // 2)],
        pebuf[0])

    copies_g = [None] * NCHUNK
    copies_w = [None] * NCHUNK
    copies_p = [None] * N_PCHUNK

    if N_PCHUNK > 1:
        copies_p[1] = pltpu.async_copy(
            pe_hbm.at[pl.ds((pos_base + CHUNK) * (D_MODEL // 2),
                            CHUNK * D_MODEL // 2)],
            pebuf[1], psem[1])

    def compute(rbuf, pbuf):
        def body(r, carry):
            for g in range(D_MODEL // 32):
                x = pbuf[pl.ds(r * (D_MODEL // 2) + g * LANES, LANES)]
                pe_a = plsc.bitcast(x << 16, jnp.float32)
                pe_b = plsc.bitcast(x & jnp.int32(-65536), jnp.float32)
                sl_a = pl.ds(g * 32, LANES)
                sl_b = pl.ds(g * 32 + LANES, LANES)
                rbuf[r, sl_a] = rbuf[r, sl_a] * SCALE + pe_a
                rbuf[r, sl_b] = rbuf[r, sl_b] * SCALE + pe_b
            return carry
        lax.fori_loop(0, CHUNK, body, 0)

    for c in range(NCHUNK + 1):
        if c < NCHUNK:
            pc, b = divmod(c, BATCH)
            slot = c % NBUF
            if c >= NBUF:
                copies_w[c - NBUF].wait()
            copies_g[c] = pltpu.async_copy(
                table_hbm.at[idx_all.at[b, pl.ds(pc * CHUNK, CHUNK)]],
                rows[slot], gsem[slot])
        if c >= 1:
            cc = c - 1
            pcc, bcc = divmod(cc, BATCH)
            cslot = cc % NBUF
            if bcc == 0 and pcc >= 1:
                copies_p[pcc].wait()
            copies_g[cc].wait()
            compute(rows[cslot], pebuf[pcc % 2])
            row_off = bcc * SEQ_LEN + pos_base + pcc * CHUNK
            copies_w[cc] = pltpu.async_copy(
                rows[cslot], out_hbm.at[pl.ds(row_off, CHUNK)], wsem[cslot])
            if bcc == BATCH - 1 and pcc + 2 < N_PCHUNK:
                npc = pcc + 2
                copies_p[npc] = pltpu.async_copy(
                    pe_hbm.at[pl.ds((pos_base + npc * CHUNK) * (D_MODEL // 2),
                                    CHUNK * D_MODEL // 2)],
                    pebuf[npc % 2], psem[npc % 2])

    for c in range(NCHUNK - NBUF, NCHUNK):
        copies_w[c].wait()


def kernel(token_ids, table):
    ids = token_ids.astype(jnp.int32)
    out = _embed(ids, table, jnp.asarray(_PE))
    return out.reshape(BATCH, SEQ_LEN, D_MODEL)


# speedup vs baseline: 1.2765x; 1.2765x over previous
import functools
import math

import jax
import jax.numpy as jnp
import numpy as np
from jax import lax
from jax.experimental import pallas as pl
from jax.experimental.pallas import tpu as pltpu
from jax.experimental.pallas import tpu_sc as plsc

VOCAB = 100000
D_MODEL = 768
BATCH = 4
SEQ_LEN = 4096
SCALE = math.sqrt(D_MODEL)

NUM_CORES = 2
NUM_SUBCORES = 16
NW = NUM_CORES * NUM_SUBCORES
POS_PER_W = SEQ_LEN // NW
CHUNK = 16
N_PCHUNK = POS_PER_W // CHUNK
NCHUNK = N_PCHUNK * BATCH
LANES = 16
WPR = D_MODEL // 2
NBUF = 4
LEAD = 2
PE_QSCALE = 16384.0
PE_INV = 1.0 / PE_QSCALE


def _pos_encoding() -> np.ndarray:
    pos = np.arange(SEQ_LEN)[:, None].astype(np.float32)
    div = np.exp(
        np.arange(0, D_MODEL, 2).astype(np.float32)
        * (-math.log(10000.0) / D_MODEL)
    )
    pe = np.zeros((SEQ_LEN, D_MODEL), dtype=np.float32)
    pe[:, 0::2] = np.sin(pos * div)
    pe[:, 1::2] = np.cos(pos * div)
    q = np.round(pe * PE_QSCALE).astype(np.int64)
    pe4 = q.reshape(SEQ_LEN, D_MODEL // 32, 2, 16)
    packed = (pe4[:, :, 0, :] & 0xFFFF) | ((pe4[:, :, 1, :] & 0xFFFF) << 16)
    return packed.astype(np.uint32).reshape(SEQ_LEN * WPR).view(np.int32)


_PE = _pos_encoding()


@functools.partial(
    pl.kernel,
    out_type=jax.ShapeDtypeStruct((BATCH * SEQ_LEN, D_MODEL), jnp.float32),
    mesh=plsc.VectorSubcoreMesh(core_axis_name="c", subcore_axis_name="s"),
    scratch_types=[
        pltpu.VMEM((BATCH * POS_PER_W,), jnp.int32),
        pltpu.VMEM((POS_PER_W * WPR,), jnp.int32),
        pltpu.VMEM((CHUNK, D_MODEL), jnp.float32),
        pltpu.VMEM((CHUNK, D_MODEL), jnp.float32),
        pltpu.VMEM((CHUNK, D_MODEL), jnp.float32),
        pltpu.VMEM((CHUNK, D_MODEL), jnp.float32),
        pltpu.SemaphoreType.DMA,
        pltpu.SemaphoreType.DMA,
        pltpu.SemaphoreType.DMA,
        pltpu.SemaphoreType.DMA,
        pltpu.SemaphoreType.DMA,
        pltpu.SemaphoreType.DMA,
        pltpu.SemaphoreType.DMA,
        pltpu.SemaphoreType.DMA,
    ],
)
def _embed(ids_hbm, table_hbm, pe_hbm, out_hbm,
           idx_all, pe_all, r0, r1, r2, r3,
           g0, g1, g2, g3, w0, w1, w2, w3):
    wid = lax.axis_index("s") * NUM_CORES + lax.axis_index("c")
    pos_base = wid * POS_PER_W

    rows = (r0, r1, r2, r3)
    gsem = (g0, g1, g2, g3)
    wsem = (w0, w1, w2, w3)

    for b in range(BATCH):
        pltpu.sync_copy(
            ids_hbm.at[pl.ds(b * SEQ_LEN + pos_base, POS_PER_W)],
            idx_all.at[pl.ds(b * POS_PER_W, POS_PER_W)])
    pltpu.sync_copy(
        pe_hbm.at[pl.ds(pos_base * WPR, POS_PER_W * WPR)], pe_all)

    def idx_slice(c):
        b = lax.rem(c, BATCH)
        pcl = lax.div(c, BATCH)
        return idx_all.at[pl.ds(b * POS_PER_W + pcl * CHUNK, CHUNK)]

    def out_slice(c):
        b = lax.rem(c, BATCH)
        pcl = lax.div(c, BATCH)
        return out_hbm.at[pl.ds(b * SEQ_LEN + pos_base + pcl * CHUNK, CHUNK)]

    def gather_start(c, slot):
        return pltpu.async_copy(table_hbm.at[idx_slice(c)],
                                rows[slot], gsem[slot])

    def compute(c, rbuf):
        pcl = lax.div(c, BATCH)
        pe_row0 = pcl * (CHUNK * WPR)

        @plsc.parallel_loop(0, CHUNK, step=1, unroll=1)
        def body(r):
            for g in range(D_MODEL // 32):
                x = pe_all[pl.ds(pe_row0 + r * WPR + g * LANES, LANES)]
                pe_a = ((x << 16) >> 16).astype(jnp.float32) * PE_INV
                pe_b = (x >> 16).astype(jnp.float32) * PE_INV
                sl_a = pl.ds(g * 32, LANES)
                sl_b = pl.ds(g * 32 + LANES, LANES)
                rbuf[r, sl_a] = rbuf[r, sl_a] * SCALE + pe_a
                rbuf[r, sl_b] = rbuf[r, sl_b] * SCALE + pe_b

    for c in range(LEAD):
        gather_start(c, c % NBUF)

    def outer(o, carry):
        for k in range(NBUF):
            c = o * NBUF + k
            nslot = (k + LEAD) % NBUF

            @pl.when(jnp.logical_and(c + LEAD < NCHUNK,
                                     c + LEAD - NBUF >= 0))
            def _():
                pltpu.make_async_copy(
                    rows[nslot], out_slice(0), wsem[nslot]).wait()

            @pl.when(c + LEAD < NCHUNK)
            def _():
                gather_start(c + LEAD, nslot)

            pltpu.make_async_copy(
                table_hbm.at[idx_slice(0)], rows[k], gsem[k]).wait()
            compute(c, rows[k])
            pltpu.async_copy(rows[k], out_slice(c), wsem[k])
        return carry

    lax.fori_loop(0, NCHUNK // NBUF, outer, 0)

    for c in range(NCHUNK - NBUF, NCHUNK):
        pltpu.make_async_copy(
            rows[c % NBUF], out_slice(0), wsem[c % NBUF]).wait()


def kernel(token_ids, table):
    ids_flat = token_ids.reshape(-1).astype(jnp.int32)
    out = _embed(ids_flat, table, jnp.asarray(_PE))
    return out.reshape(BATCH, SEQ_LEN, D_MODEL)


# speedup vs baseline: 1.3693x; 1.0727x over previous
import functools
import math

import jax
import jax.numpy as jnp
import numpy as np
from jax import lax
from jax.experimental import pallas as pl
from jax.experimental.pallas import tpu as pltpu
from jax.experimental.pallas import tpu_sc as plsc

VOCAB = 100000
D_MODEL = 768
BATCH = 4
SEQ_LEN = 4096
SCALE = math.sqrt(D_MODEL)

NUM_CORES = 2
NUM_SUBCORES = 16
NW = NUM_CORES * NUM_SUBCORES
POS_PER_W = SEQ_LEN // NW
CHUNK = 16
N_PCHUNK = POS_PER_W // CHUNK
LANES = 16
VPR = D_MODEL // LANES


def _pos_encoding() -> np.ndarray:
    pos = np.arange(SEQ_LEN)[:, None].astype(np.float32)
    div = np.exp(
        np.arange(0, D_MODEL, 2).astype(np.float32)
        * (-math.log(10000.0) / D_MODEL)
    )
    pe = np.zeros((SEQ_LEN, D_MODEL), dtype=np.float32)
    pe[:, 0::2] = np.sin(pos * div)
    pe[:, 1::2] = np.cos(pos * div)
    return pe


_PE = _pos_encoding()

_ROWS_T = pltpu.VMEM((CHUNK, D_MODEL), jnp.float32)


@functools.partial(
    pl.kernel,
    out_type=jax.ShapeDtypeStruct((BATCH * SEQ_LEN, D_MODEL), jnp.float32),
    mesh=plsc.VectorSubcoreMesh(core_axis_name="c", subcore_axis_name="s"),
    scratch_types=[
        pltpu.VMEM((BATCH, POS_PER_W), jnp.int32),
        _ROWS_T, _ROWS_T,
        _ROWS_T, _ROWS_T, _ROWS_T, _ROWS_T,
        _ROWS_T, _ROWS_T, _ROWS_T, _ROWS_T,
        pltpu.SemaphoreType.DMA,
        pltpu.SemaphoreType.DMA,
        pltpu.SemaphoreType.DMA,
        pltpu.SemaphoreType.DMA,
        pltpu.SemaphoreType.DMA,
        pltpu.SemaphoreType.DMA,
    ],
)
def _embed(ids_hbm, table_hbm, pe_hbm, out_hbm,
           idx_all, pe0, pe1,
           a0, a1, a2, a3, b0, b1, b2, b3,
           g0, g1, w0, w1, p0, p1):
    wid = lax.axis_index("s") * NUM_CORES + lax.axis_index("c")
    pos_base = wid * POS_PER_W

    groups = ((a0, a1, a2, a3), (b0, b1, b2, b3))
    gsem = (g0, g1)
    wsem = (w0, w1)
    pebuf = (pe0, pe1)
    psem = (p0, p1)

    pltpu.sync_copy(ids_hbm.at[:, pl.ds(pos_base, POS_PER_W)], idx_all)
    pltpu.sync_copy(pe_hbm.at[pl.ds(pos_base, CHUNK)], pebuf[0])
    copies_p = [None] * N_PCHUNK
    if N_PCHUNK > 1:
        copies_p[1] = pltpu.async_copy(
            pe_hbm.at[pl.ds(pos_base + CHUNK, CHUNK)], pebuf[1], psem[1])

    def gather_group(pc, g):
        cs = []
        for b in range(BATCH):
            cs.append(pltpu.async_copy(
                table_hbm.at[idx_all.at[b, pl.ds(pc * CHUNK, CHUNK)]],
                groups[g][b], gsem[g]))
        return cs

    def write_group(pc, g):
        cs = []
        for b in range(BATCH):
            row_off = b * SEQ_LEN + pos_base + pc * CHUNK
            cs.append(pltpu.async_copy(
                groups[g][b], out_hbm.at[pl.ds(row_off, CHUNK)], wsem[g]))
        return cs

    def compute(g, pbuf):
        bufs = groups[g]

        def body(r, carry):
            for j in range(VPR):
                sl = pl.ds(j * LANES, LANES)
                pe_v = pbuf[r, sl]
                for b in range(BATCH):
                    bufs[b][r, sl] = bufs[b][r, sl] * SCALE + pe_v
            return carry
        lax.fori_loop(0, CHUNK, body, 0)

    copies_g = [None] * N_PCHUNK
    copies_w = [None] * N_PCHUNK

    copies_g[0] = gather_group(0, 0)

    for pc in range(N_PCHUNK):
        g = pc % 2
        ng = (pc + 1) % 2
        if pc + 1 < N_PCHUNK:
            if pc - 1 >= 0:
                for c in copies_w[pc - 1]:
                    c.wait()
            copies_g[pc + 1] = gather_group(pc + 1, ng)
        if pc >= 1:
            copies_p[pc].wait()
        for c in copies_g[pc]:
            c.wait()
        compute(g, pebuf[pc % 2])
        copies_w[pc] = write_group(pc, g)
        if pc + 2 < N_PCHUNK:
            copies_p[pc + 2] = pltpu.async_copy(
                pe_hbm.at[pl.ds(pos_base + (pc + 2) * CHUNK, CHUNK)],
                pebuf[pc % 2], psem[pc % 2])

    for pc in (N_PCHUNK - 2, N_PCHUNK - 1):
        for c in copies_w[pc]:
            c.wait()


def kernel(token_ids, table):
    ids = token_ids.astype(jnp.int32)
    out = _embed(ids, table, jnp.asarray(_PE))
    return out.reshape(BATCH, SEQ_LEN, D_MODEL)


# speedup vs baseline: 1.4638x; 1.0691x over previous
import functools
import math

import jax
import jax.numpy as jnp
import numpy as np
from jax import lax
from jax.experimental import pallas as pl
from jax.experimental.pallas import tpu as pltpu
from jax.experimental.pallas import tpu_sc as plsc

VOCAB = 100000
D_MODEL = 768
BATCH = 4
SEQ_LEN = 4096
SCALE = math.sqrt(D_MODEL)

NUM_CORES = 2
NUM_SUBCORES = 16
NW = NUM_CORES * NUM_SUBCORES
POS_PER_W = SEQ_LEN // NW
CHUNK = 32
N_PCHUNK = POS_PER_W // CHUNK
NCHUNK = N_PCHUNK * BATCH
LANES = 16
VPR = D_MODEL // LANES
NBUF = 3


def _pos_encoding() -> np.ndarray:
    pos = np.arange(SEQ_LEN)[:, None].astype(np.float32)
    div = np.exp(
        np.arange(0, D_MODEL, 2).astype(np.float32)
        * (-math.log(10000.0) / D_MODEL)
    )
    pe = np.zeros((SEQ_LEN, D_MODEL), dtype=np.float32)
    pe[:, 0::2] = np.sin(pos * div)
    pe[:, 1::2] = np.cos(pos * div)
    return pe


_PE = _pos_encoding()


@functools.partial(
    pl.kernel,
    out_type=jax.ShapeDtypeStruct((BATCH * SEQ_LEN, D_MODEL), jnp.float32),
    mesh=plsc.VectorSubcoreMesh(core_axis_name="c", subcore_axis_name="s"),
    scratch_types=[
        pltpu.VMEM((BATCH, POS_PER_W), jnp.int32),
        pltpu.VMEM((CHUNK, D_MODEL), jnp.float32),
        pltpu.VMEM((CHUNK, D_MODEL), jnp.float32),
        pltpu.VMEM((CHUNK, D_MODEL), jnp.float32),
        pltpu.VMEM((CHUNK, D_MODEL), jnp.float32),
        pltpu.VMEM((CHUNK, D_MODEL), jnp.float32),
        pltpu.SemaphoreType.DMA,
        pltpu.SemaphoreType.DMA,
        pltpu.SemaphoreType.DMA,
        pltpu.SemaphoreType.DMA,
        pltpu.SemaphoreType.DMA,
        pltpu.SemaphoreType.DMA,
        pltpu.SemaphoreType.DMA,
        pltpu.SemaphoreType.DMA,
    ],
)
def _embed(ids_hbm, table_hbm, pe_hbm, out_hbm,
           idx_all, pe0, pe1, r0, r1, r2,
           g0, g1, g2, w0, w1, w2, p0, p1):
    wid = lax.axis_index("s") * NUM_CORES + lax.axis_index("c")
    pos_base = wid * POS_PER_W

    rows = (r0, r1, r2)
    gsem = (g0, g1, g2)
    wsem = (w0, w1, w2)
    pebuf = (pe0, pe1)
    psem = (p0, p1)

    pltpu.sync_copy(ids_hbm.at[:, pl.ds(pos_base, POS_PER_W)], idx_all)
    pltpu.sync_copy(pe_hbm.at[pl.ds(pos_base, CHUNK)], pebuf[0])

    copies_g = [None] * NCHUNK
    copies_w = [None] * NCHUNK
    copies_p = [None] * N_PCHUNK

    if N_PCHUNK > 1:
        copies_p[1] = pltpu.async_copy(
            pe_hbm.at[pl.ds(pos_base + CHUNK, CHUNK)], pebuf[1], psem[1])

    def compute(rbuf, pbuf):
        def body(r, carry):
            for j in range(VPR):
                sl = pl.ds(j * LANES, LANES)
                rbuf[r, sl] = rbuf[r, sl] * SCALE + pbuf[r, sl]
            return carry
        lax.fori_loop(0, CHUNK, body, 0)

    for c in range(NCHUNK + 1):
        if c < NCHUNK:
            pc, b = divmod(c, BATCH)
            slot = c % NBUF
            if c >= NBUF:
                copies_w[c - NBUF].wait()
            copies_g[c] = pltpu.async_copy(
                table_hbm.at[idx_all.at[b, pl.ds(pc * CHUNK, CHUNK)]],
                rows[slot], gsem[slot])
        if c >= 1:
            cc = c - 1
            pcc, bcc = divmod(cc, BATCH)
            cslot = cc % NBUF
            if bcc == 0 and pcc >= 1:
                copies_p[pcc].wait()
            copies_g[cc].wait()
            compute(rows[cslot], pebuf[pcc % 2])
            row_off = bcc * SEQ_LEN + pos_base + pcc * CHUNK
            copies_w[cc] = pltpu.async_copy(
                rows[cslot], out_hbm.at[pl.ds(row_off, CHUNK)], wsem[cslot])
            if bcc == BATCH - 1 and pcc + 2 < N_PCHUNK:
                npc = pcc + 2
                copies_p[npc] = pltpu.async_copy(
                    pe_hbm.at[pl.ds(pos_base + npc * CHUNK, CHUNK)],
                    pebuf[npc % 2], psem[npc % 2])

    for c in range(NCHUNK - NBUF, NCHUNK):
        copies_w[c].wait()


def kernel(token_ids, table):
    ids = token_ids.astype(jnp.int32)
    out = _embed(ids, table, jnp.asarray(_PE))
    return out.reshape(BATCH, SEQ_LEN, D_MODEL)
